# baseline (speedup 1.0000x reference)
import functools

import jax
import jax.numpy as jnp
from jax import lax
from jax.experimental import pallas as pl
from jax.experimental.pallas import tpu as pltpu
from jax.experimental.pallas import tpu_sc as plsc

N_NODES = 10000
N_EDGES = 160000
IN_DIM = 256
HID = 256
HEADS = 4

NC = 2
NS = 16
LANES = 16

N_PAD = 10240
E_REAL = N_EDGES + N_NODES
CHUNK = 64
E_TEC = 10752
E_PAD = E_TEC * NS
NCH = E_TEC // CHUNK
ROWS_TEC = N_PAD // NS
HALF = 128

_BLK = 1024


def _proj_body(x_ref, w_ref, asrc_ref, adst_ref, h_ref, a_src_ref, a_dst_ref):
    h = jnp.dot(x_ref[...], w_ref[...], preferred_element_type=jnp.float32)
    h_ref[...] = h
    a_src_ref[...] = jnp.dot(h, asrc_ref[...], preferred_element_type=jnp.float32)
    a_dst_ref[...] = jnp.dot(h, adst_ref[...], preferred_element_type=jnp.float32)


def _project(x, W, A_src, A_dst, heads):
    n, k = x.shape
    f = W.shape[1]
    return pl.pallas_call(
        _proj_body,
        grid=(n // _BLK,),
        in_specs=[
            pl.BlockSpec((_BLK, k), lambda i: (i, 0)),
            pl.BlockSpec((k, f), lambda i: (0, 0)),
            pl.BlockSpec((f, heads), lambda i: (0, 0)),
            pl.BlockSpec((f, heads), lambda i: (0, 0)),
        ],
        out_specs=[
            pl.BlockSpec((_BLK, f), lambda i: (i, 0)),
            pl.BlockSpec((_BLK, heads), lambda i: (i, 0)),
            pl.BlockSpec((_BLK, heads), lambda i: (i, 0)),
        ],
        out_shape=[
            jax.ShapeDtypeStruct((n, f), jnp.float32),
            jax.ShapeDtypeStruct((n, heads), jnp.float32),
            jax.ShapeDtypeStruct((n, heads), jnp.float32),
        ],
    )(x, W, A_src, A_dst)


def _proj2_body(o1_ref, b1_ref, w2_ref, ws_ref, wd_ref,
                h2_ref, a_src_ref, a_dst_ref):
    k = pl.program_id(1)
    v = o1_ref[0] + b1_ref[0]
    hmid = jnp.where(v > 0, v, jnp.exp(v) - 1.0)
    ph = jnp.dot(hmid, w2_ref[0], preferred_element_type=jnp.float32)
    ps = jnp.dot(hmid, ws_ref[0], preferred_element_type=jnp.float32)
    pd = jnp.dot(hmid, wd_ref[0], preferred_element_type=jnp.float32)

    @pl.when(k == 0)
    def _():
        h2_ref[...] = ph
        a_src_ref[...] = ps
        a_dst_ref[...] = pd

    @pl.when(k > 0)
    def _():
        h2_ref[...] += ph
        a_src_ref[...] += ps
        a_dst_ref[...] += pd


def _project2(out1_flat, bias1, W2, watt_s, watt_d):
    nk = out1_flat.shape[0]
    return pl.pallas_call(
        _proj2_body,
        grid=(N_PAD // _BLK, nk),
        in_specs=[
            pl.BlockSpec((1, _BLK, HALF), lambda i, k: (k, i, 0)),
            pl.BlockSpec((1, 1, HALF), lambda i, k: (k, 0, 0)),
            pl.BlockSpec((1, HALF, HID), lambda i, k: (k, 0, 0)),
            pl.BlockSpec((1, HALF, 1), lambda i, k: (k, 0, 0)),
            pl.BlockSpec((1, HALF, 1), lambda i, k: (k, 0, 0)),
        ],
        out_specs=[
            pl.BlockSpec((_BLK, HID), lambda i, k: (i, 0)),
            pl.BlockSpec((_BLK, 1), lambda i, k: (i, 0)),
            pl.BlockSpec((_BLK, 1), lambda i, k: (i, 0)),
        ],
        out_shape=[
            jax.ShapeDtypeStruct((N_PAD, HID), jnp.float32),
            jax.ShapeDtypeStruct((N_PAD, 1), jnp.float32),
            jax.ShapeDtypeStruct((N_PAD, 1), jnp.float32),
        ],
    )(out1_flat, bias1.reshape(nk, 1, HALF), W2.reshape(nk, HALF, HID),
      watt_s.reshape(nk, HALF, 1), watt_d.reshape(nk, HALF, 1))


def _gat_sc_body(heads,
                 idx_hbm, asrcT_hbm, adstT_hbm, hflat_hbm,
                 out_hbm,
                 asrc_t, adst_t, den_s, rbf0, rbf1, rowsf0, rowsf1,
                 idxc0, idxc1, eac0, eac1,
                 sem_i0, sem_i1, sem_g0, sem_g1,
                 sem_s0, sem_s1, sem_d0, sem_d1,
                 sh_out, sh_den):
    csc = lax.axis_index("c")
    s = lax.axis_index("s")
    cbase = s * NCH
    row0 = s * ROWS_TEC

    zero16 = jnp.zeros((LANES,), jnp.float32)
    rbf = (rbf0, rbf1)
    rowsf = (rowsf0, rowsf1)
    idxc = (idxc0, idxc1)
    eac = (eac0, eac1)
    sem_i = (sem_i0, sem_i1)
    sem_g = (sem_g0, sem_g1)
    sem_s = (sem_s0, sem_s1)
    sem_d = (sem_d0, sem_d1)

    def issue_idx(c, b):
        pltpu.async_copy(idx_hbm.at[cbase + c], idxc[b], sem_i[b])

    def wait_idx(b):
        pltpu.make_async_copy(idx_hbm.at[cbase], idxc[b], sem_i[b]).wait()

    def issue_gather(b, u):
        pltpu.async_copy(hflat_hbm.at[u].at[idxc[b].at[0]], rbf[b], sem_g[b])

    def wait_gather(b, u):
        pltpu.make_async_copy(hflat_hbm.at[u].at[idxc[b].at[0]], rbf[b],
                              sem_g[b]).wait()

    def issue_scat(b):
        pltpu.async_copy(rowsf[b], sh_out.at[idxc[b].at[1]], sem_s[b],
                         add=True)

    def wait_scat(b):
        pltpu.make_async_copy(rowsf[b], sh_out.at[idxc[b].at[1]],
                              sem_s[b]).wait()

    def issue_den(b):
        pltpu.async_copy(eac[b], sh_den.at[idxc[b].at[1]], sem_d[b], add=True)

    def wait_den(b):
        pltpu.make_async_copy(eac[b], sh_den.at[idxc[b].at[1]],
                              sem_d[b]).wait()

    for hd in range(heads):
        u = hd * NC + csc

        def zrow(i, _):
            for j in range(HALF // LANES):
                rowsf0[i, pl.ds(j * LANES, LANES)] = zero16
            return 0
        lax.fori_loop(0, CHUNK, zrow, 0)
        for j in range(CHUNK // LANES):
            eac0[pl.ds(j * LANES, LANES)] = zero16
        for z in range(ROWS_TEC // CHUNK):
            pltpu.sync_copy(rowsf0, sh_out.at[pl.ds(row0 + z * CHUNK, CHUNK), :])
            pltpu.sync_copy(eac0, sh_den.at[pl.ds(row0 + z * CHUNK, CHUNK)])

        pltpu.sync_copy(asrcT_hbm.at[hd], asrc_t)
        pltpu.sync_copy(adstT_hbm.at[hd], adst_t)
        plsc.subcore_barrier()

        def logits(b):
            for j in range(CHUNK // LANES):
                jl = pl.ds(j * LANES, LANES)
                a = plsc.load_gather(asrc_t, [idxc[b][0, jl]])
                bl = plsc.load_gather(adst_t, [idxc[b][1, jl]])
                al = a + bl
                al = jnp.where(al >= 0.0, al, 0.2 * al)
                eac[b][jl] = jnp.exp(al)

        def scale(b):
            @plsc.parallel_loop(0, CHUNK, unroll=4)
            def _srow(r):
                av = plsc.load_gather(eac[b],
                                      [jnp.full((LANES,), r, jnp.int32)])
                for j in range(HALF // 32):
                    v = rbf[b][r, pl.ds(j * 32, 32)]
                    lo, hi = plsc.unpack(v, format=plsc.PackFormat.INTERLEAVED)
                    rowsf[b][r, pl.ds(j * 32, LANES)] = lo * av
                    rowsf[b][r, pl.ds(j * 32 + LANES, LANES)] = hi * av

        def half(c, b, pred_w, pred_n, pred_tail):
            b1 = 1 - b

            @pl.when(pred_w)
            def _():
                wait_den(b)
            logits(b)
            issue_den(b)

            @pl.when(pred_n)
            def _():
                wait_idx(b1)
                issue_gather(b1, u)

            wait_gather(b, u)

            @pl.when(pred_w)
            def _():
                wait_scat(b)
            scale(b)
            issue_scat(b)

            @pl.when(pred_tail)
            def _():
                issue_idx(c + 2, b)

        issue_idx(0, 0)
        wait_idx(0)
        issue_gather(0, u)
        issue_idx(1, 1)

        npair = NCH // 2

        def step(i, _):
            true_ = i >= 0
            last = npair - 1
            half(2 * i, 0, i >= 1, true_, i < last)
            half(2 * i + 1, 1, i >= 1, i < last, i < last)
            return 0
        lax.fori_loop(0, npair, step, 0)

        wait_scat(0)
        wait_scat(1)
        wait_den(0)
        wait_den(1)
        plsc.subcore_barrier()

        pltpu.sync_copy(sh_den.at[pl.ds(row0, ROWS_TEC)], den_s)

        def flush(z, _):
            r0 = row0 + z * CHUNK
            pltpu.sync_copy(sh_out.at[pl.ds(r0, CHUNK), :], rowsf0)

            @plsc.parallel_loop(0, CHUNK, unroll=4)
            def _norm(r):
                dv = plsc.load_gather(den_s, [jnp.full((LANES,),
                                                       z * CHUNK + r,
                                                       jnp.int32)])
                inv = 1.0 / (dv + 1e-16)
                for j in range(HALF // LANES):
                    jl = pl.ds(j * LANES, LANES)
                    rowsf0[r, jl] = rowsf0[r, jl] * inv
            pltpu.sync_copy(rowsf0, out_hbm.at[hd, csc, pl.ds(r0, CHUNK), :])
            return 0
        lax.fori_loop(0, ROWS_TEC // CHUNK, flush, 0)
        plsc.subcore_barrier()


def _gat_sc(idx_packed, asrcT, adstT, h_flat, heads):
    mesh = plsc.VectorSubcoreMesh(core_axis_name="c", subcore_axis_name="s",
                                  num_cores=NC, num_subcores=NS)
    return pl.kernel(
        functools.partial(_gat_sc_body, heads),
        out_type=jax.ShapeDtypeStruct((heads, NC, N_PAD, HALF), jnp.float32),
        mesh=mesh,
        compiler_params=pltpu.CompilerParams(needs_layout_passes=False, use_tc_tiling_on_sc=False),
        scratch_types=[
            pltpu.VMEM((N_PAD,), jnp.float32),
            pltpu.VMEM((N_PAD,), jnp.float32),
            pltpu.VMEM((ROWS_TEC,), jnp.float32),
            pltpu.VMEM((CHUNK, HALF), jnp.bfloat16),
            pltpu.VMEM((CHUNK, HALF), jnp.bfloat16),
            pltpu.VMEM((CHUNK, HALF), jnp.float32),
            pltpu.VMEM((CHUNK, HALF), jnp.float32),
            pltpu.VMEM((2, CHUNK), jnp.int32),
            pltpu.VMEM((2, CHUNK), jnp.int32),
            pltpu.VMEM((CHUNK,), jnp.float32),
            pltpu.VMEM((CHUNK,), jnp.float32),
            pltpu.SemaphoreType.DMA,
            pltpu.SemaphoreType.DMA,
            pltpu.SemaphoreType.DMA,
            pltpu.SemaphoreType.DMA,
            pltpu.SemaphoreType.DMA,
            pltpu.SemaphoreType.DMA,
            pltpu.SemaphoreType.DMA,
            pltpu.SemaphoreType.DMA,
            pltpu.VMEM_SHARED((N_PAD, HALF), jnp.float32),
            pltpu.VMEM_SHARED((N_PAD,), jnp.float32),
        ],
    )(idx_packed, asrcT, adstT, h_flat)


def _pack_cols(h_flat):
    sh = h_flat.shape
    v = h_flat.reshape(sh[:-1] + (sh[-1] // 32, 2, 16))
    v = v.swapaxes(-1, -2).reshape(sh)
    return v.astype(jnp.bfloat16)


def _blockdiag(att, heads, d):
    eye = jnp.eye(heads, dtype=jnp.float32)
    return (att.reshape(heads, 1, d) * eye[:, :, None]).transpose(0, 2, 1).reshape(heads * d, heads)


def kernel(x, edge_index, W1, att_src1, att_dst1, bias1, W2, att_src2, att_dst2, bias2):
    idt = edge_index.dtype
    loop = jnp.arange(N_NODES, dtype=idt)
    n_pad_e = E_PAD - E_REAL
    pad_src = jnp.zeros((n_pad_e,), dtype=idt)
    pad_dst = (N_NODES + jnp.arange(n_pad_e, dtype=idt) % (N_PAD - N_NODES))
    src = jnp.concatenate([edge_index[0], loop, pad_src]).astype(jnp.int32)
    dst = jnp.concatenate([edge_index[1], loop, pad_dst]).astype(jnp.int32)
    idx_packed = jnp.stack([src.reshape(-1, CHUNK), dst.reshape(-1, CHUNK)],
                           axis=1)

    A_src1 = _blockdiag(att_src1, HEADS, HID)
    A_dst1 = _blockdiag(att_dst1, HEADS, HID)

    x_pad = jnp.pad(x, ((0, N_PAD - N_NODES), (0, 0)))

    h1, as1, ad1 = _project(x_pad, W1, A_src1, A_dst1, HEADS)
    h1_flat = _pack_cols(h1.reshape(N_PAD, HEADS, NC, HALF)
                           .transpose(1, 2, 0, 3).reshape(HEADS * NC, N_PAD, HALF))
    out1 = _gat_sc(idx_packed, as1.T, ad1.T, h1_flat, HEADS)

    watt_s = W2 @ att_src2.reshape(HID, 1)
    watt_d = W2 @ att_dst2.reshape(HID, 1)
    out1_flat = out1.reshape(HEADS * NC, N_PAD, HALF)
    h2, as2, ad2 = _project2(out1_flat, bias1, W2, watt_s, watt_d)
    h2_flat = _pack_cols(h2.reshape(N_PAD, NC, HALF).transpose(1, 0, 2))
    out2 = _gat_sc(idx_packed, as2.T, ad2.T, h2_flat, 1)

    out = jnp.concatenate([out2[0, 0, :N_NODES], out2[0, 1, :N_NODES]], axis=1)
    return out + bias2


# speedup vs baseline: 1.0431x; 1.0431x over previous
import functools

import jax
import jax.numpy as jnp
from jax import lax
from jax.experimental import pallas as pl
from jax.experimental.pallas import tpu as pltpu
from jax.experimental.pallas import tpu_sc as plsc

N_NODES = 10000
N_EDGES = 160000
IN_DIM = 256
HID = 256
HEADS = 4

NC = 2
NS = 16
LANES = 16

N_PAD = 10240
E_REAL = N_EDGES + N_NODES
CHUNK = 64
E_TEC = 10752
E_PAD = E_TEC * NS
NCH = E_TEC // CHUNK
ROWS_TEC = N_PAD // NS
HALF = 128

_BLK = 1024


def _proj_body(x_ref, w_ref, asrc_ref, adst_ref, h_ref, a_src_ref, a_dst_ref):
    h = jnp.dot(x_ref[...], w_ref[...], preferred_element_type=jnp.float32)
    h_ref[...] = h
    a_src_ref[...] = jnp.dot(h, asrc_ref[...], preferred_element_type=jnp.float32)
    a_dst_ref[...] = jnp.dot(h, adst_ref[...], preferred_element_type=jnp.float32)


def _project(x, W, A_src, A_dst, heads):
    n, k = x.shape
    f = W.shape[1]
    return pl.pallas_call(
        _proj_body,
        grid=(n // _BLK,),
        in_specs=[
            pl.BlockSpec((_BLK, k), lambda i: (i, 0)),
            pl.BlockSpec((k, f), lambda i: (0, 0)),
            pl.BlockSpec((f, heads), lambda i: (0, 0)),
            pl.BlockSpec((f, heads), lambda i: (0, 0)),
        ],
        out_specs=[
            pl.BlockSpec((_BLK, f), lambda i: (i, 0)),
            pl.BlockSpec((_BLK, heads), lambda i: (i, 0)),
            pl.BlockSpec((_BLK, heads), lambda i: (i, 0)),
        ],
        out_shape=[
            jax.ShapeDtypeStruct((n, f), jnp.float32),
            jax.ShapeDtypeStruct((n, heads), jnp.float32),
            jax.ShapeDtypeStruct((n, heads), jnp.float32),
        ],
    )(x, W, A_src, A_dst)


def _proj2_body(o1_ref, b1_ref, w2_ref, ws_ref, wd_ref,
                h2_ref, a_src_ref, a_dst_ref):
    k = pl.program_id(1)
    v = o1_ref[0] + b1_ref[0]
    hmid = jnp.where(v > 0, v, jnp.exp(v) - 1.0)
    ph = jnp.dot(hmid, w2_ref[0], preferred_element_type=jnp.float32)
    ps = jnp.dot(hmid, ws_ref[0], preferred_element_type=jnp.float32)
    pd = jnp.dot(hmid, wd_ref[0], preferred_element_type=jnp.float32)

    @pl.when(k == 0)
    def _():
        h2_ref[...] = ph
        a_src_ref[...] = ps
        a_dst_ref[...] = pd

    @pl.when(k > 0)
    def _():
        h2_ref[...] += ph
        a_src_ref[...] += ps
        a_dst_ref[...] += pd


def _project2(out1_flat, bias1, W2, watt_s, watt_d):
    nk = out1_flat.shape[0]
    return pl.pallas_call(
        _proj2_body,
        grid=(N_PAD // _BLK, nk),
        in_specs=[
            pl.BlockSpec((1, _BLK, HALF), lambda i, k: (k, i, 0)),
            pl.BlockSpec((1, 1, HALF), lambda i, k: (k, 0, 0)),
            pl.BlockSpec((1, HALF, HID), lambda i, k: (k, 0, 0)),
            pl.BlockSpec((1, HALF, 1), lambda i, k: (k, 0, 0)),
            pl.BlockSpec((1, HALF, 1), lambda i, k: (k, 0, 0)),
        ],
        out_specs=[
            pl.BlockSpec((_BLK, HID), lambda i, k: (i, 0)),
            pl.BlockSpec((_BLK, 1), lambda i, k: (i, 0)),
            pl.BlockSpec((_BLK, 1), lambda i, k: (i, 0)),
        ],
        out_shape=[
            jax.ShapeDtypeStruct((N_PAD, HID), jnp.float32),
            jax.ShapeDtypeStruct((N_PAD, 1), jnp.float32),
            jax.ShapeDtypeStruct((N_PAD, 1), jnp.float32),
        ],
    )(out1_flat, bias1.reshape(nk, 1, HALF), W2.reshape(nk, HALF, HID),
      watt_s.reshape(nk, HALF, 1), watt_d.reshape(nk, HALF, 1))


def _gat_sc_body(heads,
                 idx_hbm, asrcT_hbm, adstT_hbm, hflat_hbm,
                 out_hbm,
                 asrc_t, adst_t, den_s, rbf0, rbf1, rowsf0, rowsf1,
                 idxc0, idxc1, eac0, eac1,
                 sem_i0, sem_i1, sem_g0, sem_g1,
                 sem_s0, sem_s1, sem_d0, sem_d1,
                 sh_out, sh_den):
    csc = lax.axis_index("c")
    s = lax.axis_index("s")
    cbase = s * NCH
    row0 = s * ROWS_TEC

    zero16 = jnp.zeros((LANES,), jnp.float32)
    rbf = (rbf0, rbf1)
    rowsf = (rowsf0, rowsf1)
    idxc = (idxc0, idxc1)
    eac = (eac0, eac1)
    sem_i = (sem_i0, sem_i1)
    sem_g = (sem_g0, sem_g1)
    sem_s = (sem_s0, sem_s1)
    sem_d = (sem_d0, sem_d1)

    def issue_idx(c, b):
        pltpu.async_copy(idx_hbm.at[cbase + c], idxc[b], sem_i[b])

    def wait_idx(b):
        pltpu.make_async_copy(idx_hbm.at[cbase], idxc[b], sem_i[b]).wait()

    def issue_gather(b, u):
        pltpu.async_copy(hflat_hbm.at[u].at[idxc[b].at[0]], rbf[b], sem_g[b])

    def wait_gather(b, u):
        pltpu.make_async_copy(hflat_hbm.at[u].at[idxc[b].at[0]], rbf[b],
                              sem_g[b]).wait()

    def issue_scat(b):
        pltpu.async_copy(rowsf[b], sh_out.at[idxc[b].at[1]], sem_s[b],
                         add=True)

    def wait_scat(b):
        pltpu.make_async_copy(rowsf[b], sh_out.at[idxc[b].at[1]],
                              sem_s[b]).wait()

    def issue_den(b):
        pltpu.async_copy(eac[b], sh_den.at[idxc[b].at[1]], sem_d[b], add=True)

    def wait_den(b):
        pltpu.make_async_copy(eac[b], sh_den.at[idxc[b].at[1]],
                              sem_d[b]).wait()

    for hd in range(heads):
        u = hd * NC + csc

        def zrow(i, _):
            for j in range(HALF // LANES):
                rowsf0[i, pl.ds(j * LANES, LANES)] = zero16
            return 0
        lax.fori_loop(0, CHUNK, zrow, 0)
        for j in range(CHUNK // LANES):
            eac0[pl.ds(j * LANES, LANES)] = zero16
        for z in range(ROWS_TEC // CHUNK):
            pltpu.sync_copy(rowsf0, sh_out.at[pl.ds(row0 + z * CHUNK, CHUNK), :])
            pltpu.sync_copy(eac0, sh_den.at[pl.ds(row0 + z * CHUNK, CHUNK)])

        pltpu.sync_copy(asrcT_hbm.at[hd], asrc_t)
        pltpu.sync_copy(adstT_hbm.at[hd], adst_t)
        plsc.subcore_barrier()

        def logits(b):
            for j in range(CHUNK // LANES):
                jl = pl.ds(j * LANES, LANES)
                a = plsc.load_gather(asrc_t, [idxc[b][0, jl]])
                bl = plsc.load_gather(adst_t, [idxc[b][1, jl]])
                al = a + bl
                al = jnp.where(al >= 0.0, al, 0.2 * al)
                eac[b][jl] = jnp.exp(al)

        def scale(b):
            @plsc.parallel_loop(0, CHUNK, unroll=4)
            def _srow(r):
                av = plsc.load_gather(eac[b],
                                      [jnp.full((LANES,), r, jnp.int32)])
                for j in range(HALF // 32):
                    vi = rbf[b][r, pl.ds(j * LANES, LANES)]
                    v = plsc.bitcast(vi, jnp.bfloat16)
                    lo, hi = plsc.unpack(v, format=plsc.PackFormat.INTERLEAVED)
                    rowsf[b][r, pl.ds(j * 32, LANES)] = lo * av
                    rowsf[b][r, pl.ds(j * 32 + LANES, LANES)] = hi * av

        def half(c, b, pred_w, pred_n, pred_tail):
            b1 = 1 - b

            @pl.when(pred_w)
            def _():
                wait_den(b)
            logits(b)
            issue_den(b)

            @pl.when(pred_n)
            def _():
                wait_idx(b1)
                issue_gather(b1, u)

            wait_gather(b, u)

            @pl.when(pred_w)
            def _():
                wait_scat(b)
            scale(b)
            issue_scat(b)

            @pl.when(pred_tail)
            def _():
                issue_idx(c + 2, b)

        issue_idx(0, 0)
        wait_idx(0)
        issue_gather(0, u)
        issue_idx(1, 1)

        npair = NCH // 2

        def step(i, _):
            true_ = i >= 0
            last = npair - 1
            half(2 * i, 0, i >= 1, true_, i < last)
            half(2 * i + 1, 1, i >= 1, i < last, i < last)
            return 0
        lax.fori_loop(0, npair, step, 0)

        wait_scat(0)
        wait_scat(1)
        wait_den(0)
        wait_den(1)
        plsc.subcore_barrier()

        pltpu.sync_copy(sh_den.at[pl.ds(row0, ROWS_TEC)], den_s)

        def flush(z, _):
            r0 = row0 + z * CHUNK
            pltpu.sync_copy(sh_out.at[pl.ds(r0, CHUNK), :], rowsf0)

            @plsc.parallel_loop(0, CHUNK, unroll=4)
            def _norm(r):
                dv = plsc.load_gather(den_s, [jnp.full((LANES,),
                                                       z * CHUNK + r,
                                                       jnp.int32)])
                inv = 1.0 / (dv + 1e-16)
                for j in range(HALF // LANES):
                    jl = pl.ds(j * LANES, LANES)
                    rowsf0[r, jl] = rowsf0[r, jl] * inv
            pltpu.sync_copy(rowsf0, out_hbm.at[hd, csc, pl.ds(r0, CHUNK), :])
            return 0
        lax.fori_loop(0, ROWS_TEC // CHUNK, flush, 0)
        plsc.subcore_barrier()


def _gat_sc(idx_packed, asrcT, adstT, h_flat, heads):
    mesh = plsc.VectorSubcoreMesh(core_axis_name="c", subcore_axis_name="s",
                                  num_cores=NC, num_subcores=NS)
    return pl.kernel(
        functools.partial(_gat_sc_body, heads),
        out_type=jax.ShapeDtypeStruct((heads, NC, N_PAD, HALF), jnp.float32),
        mesh=mesh,
        compiler_params=pltpu.CompilerParams(needs_layout_passes=False, use_tc_tiling_on_sc=False),
        scratch_types=[
            pltpu.VMEM((N_PAD,), jnp.float32),
            pltpu.VMEM((N_PAD,), jnp.float32),
            pltpu.VMEM((ROWS_TEC,), jnp.float32),
            pltpu.VMEM((CHUNK, HALF // 2), jnp.int32),
            pltpu.VMEM((CHUNK, HALF // 2), jnp.int32),
            pltpu.VMEM((CHUNK, HALF), jnp.float32),
            pltpu.VMEM((CHUNK, HALF), jnp.float32),
            pltpu.VMEM((2, CHUNK), jnp.int32),
            pltpu.VMEM((2, CHUNK), jnp.int32),
            pltpu.VMEM((CHUNK,), jnp.float32),
            pltpu.VMEM((CHUNK,), jnp.float32),
            pltpu.SemaphoreType.DMA,
            pltpu.SemaphoreType.DMA,
            pltpu.SemaphoreType.DMA,
            pltpu.SemaphoreType.DMA,
            pltpu.SemaphoreType.DMA,
            pltpu.SemaphoreType.DMA,
            pltpu.SemaphoreType.DMA,
            pltpu.SemaphoreType.DMA,
            pltpu.VMEM_SHARED((N_PAD, HALF), jnp.float32),
            pltpu.VMEM_SHARED((N_PAD,), jnp.float32),
        ],
    )(idx_packed, asrcT, adstT, h_flat)


def _pack_cols(h_flat):
    sh = h_flat.shape
    v = h_flat.reshape(sh[:-1] + (sh[-1] // 32, 2, 16))
    v = v.swapaxes(-1, -2).reshape(sh[:-1] + (sh[-1] // 2, 2))
    return lax.bitcast_convert_type(v.astype(jnp.bfloat16), jnp.int32)


def _blockdiag(att, heads, d):
    eye = jnp.eye(heads, dtype=jnp.float32)
    return (att.reshape(heads, 1, d) * eye[:, :, None]).transpose(0, 2, 1).reshape(heads * d, heads)


def kernel(x, edge_index, W1, att_src1, att_dst1, bias1, W2, att_src2, att_dst2, bias2):
    idt = edge_index.dtype
    loop = jnp.arange(N_NODES, dtype=idt)
    n_pad_e = E_PAD - E_REAL
    pad_src = jnp.zeros((n_pad_e,), dtype=idt)
    pad_dst = (N_NODES + jnp.arange(n_pad_e, dtype=idt) % (N_PAD - N_NODES))
    src = jnp.concatenate([edge_index[0], loop, pad_src]).astype(jnp.int32)
    dst = jnp.concatenate([edge_index[1], loop, pad_dst]).astype(jnp.int32)
    idx_packed = jnp.stack([src.reshape(-1, CHUNK), dst.reshape(-1, CHUNK)],
                           axis=1)

    A_src1 = _blockdiag(att_src1, HEADS, HID)
    A_dst1 = _blockdiag(att_dst1, HEADS, HID)

    x_pad = jnp.pad(x, ((0, N_PAD - N_NODES), (0, 0)))

    h1, as1, ad1 = _project(x_pad, W1, A_src1, A_dst1, HEADS)
    h1_flat = _pack_cols(h1.reshape(N_PAD, HEADS, NC, HALF)
                           .transpose(1, 2, 0, 3).reshape(HEADS * NC, N_PAD, HALF))
    out1 = _gat_sc(idx_packed, as1.T, ad1.T, h1_flat, HEADS)

    watt_s = W2 @ att_src2.reshape(HID, 1)
    watt_d = W2 @ att_dst2.reshape(HID, 1)
    out1_flat = out1.reshape(HEADS * NC, N_PAD, HALF)
    h2, as2, ad2 = _project2(out1_flat, bias1, W2, watt_s, watt_d)
    h2_flat = _pack_cols(h2.reshape(N_PAD, NC, HALF).transpose(1, 0, 2))
    out2 = _gat_sc(idx_packed, as2.T, ad2.T, h2_flat, 1)

    out = jnp.concatenate([out2[0, 0, :N_NODES], out2[0, 1, :N_NODES]], axis=1)
    return out + bias2


# speedup vs baseline: 1.0435x; 1.0004x over previous
import functools

import jax
import jax.numpy as jnp
from jax import lax
from jax.experimental import pallas as pl
from jax.experimental.pallas import tpu as pltpu
from jax.experimental.pallas import tpu_sc as plsc

N_NODES = 10000
N_EDGES = 160000
IN_DIM = 256
HID = 256
HEADS = 4

NC = 2
NS = 16
LANES = 16

N_PAD = 10240
E_REAL = N_EDGES + N_NODES
CHUNK = 64
E_TEC = 10752
E_PAD = E_TEC * NS
NCH = E_TEC // CHUNK
ROWS_TEC = N_PAD // NS
HALF = 128

_BLK = 1024


def _proj_body(x_ref, w_ref, asrc_ref, adst_ref, h_ref, a_src_ref, a_dst_ref):
    h = jnp.dot(x_ref[...].astype(jnp.bfloat16), w_ref[...].astype(jnp.bfloat16),
                preferred_element_type=jnp.float32)
    h_ref[...] = h
    a_src_ref[...] = jnp.dot(h, asrc_ref[...], preferred_element_type=jnp.float32)
    a_dst_ref[...] = jnp.dot(h, adst_ref[...], preferred_element_type=jnp.float32)


def _project(x, W, A_src, A_dst, heads):
    n, k = x.shape
    f = W.shape[1]
    return pl.pallas_call(
        _proj_body,
        grid=(n // _BLK,),
        in_specs=[
            pl.BlockSpec((_BLK, k), lambda i: (i, 0)),
            pl.BlockSpec((k, f), lambda i: (0, 0)),
            pl.BlockSpec((f, heads), lambda i: (0, 0)),
            pl.BlockSpec((f, heads), lambda i: (0, 0)),
        ],
        out_specs=[
            pl.BlockSpec((_BLK, f), lambda i: (i, 0)),
            pl.BlockSpec((_BLK, heads), lambda i: (i, 0)),
            pl.BlockSpec((_BLK, heads), lambda i: (i, 0)),
        ],
        out_shape=[
            jax.ShapeDtypeStruct((n, f), jnp.float32),
            jax.ShapeDtypeStruct((n, heads), jnp.float32),
            jax.ShapeDtypeStruct((n, heads), jnp.float32),
        ],
    )(x, W, A_src, A_dst)


def _proj2_body(o1_ref, b1_ref, w2_ref, ws_ref, wd_ref,
                h2_ref, a_src_ref, a_dst_ref):
    k = pl.program_id(1)
    v = o1_ref[0] + b1_ref[0]
    hmid = jnp.where(v > 0, v, jnp.exp(v) - 1.0)
    hb = hmid.astype(jnp.bfloat16)
    ph = jnp.dot(hb, w2_ref[0].astype(jnp.bfloat16),
                 preferred_element_type=jnp.float32)
    ps = jnp.dot(hb, ws_ref[0].astype(jnp.bfloat16),
                 preferred_element_type=jnp.float32)
    pd = jnp.dot(hb, wd_ref[0].astype(jnp.bfloat16),
                 preferred_element_type=jnp.float32)

    @pl.when(k == 0)
    def _():
        h2_ref[...] = ph
        a_src_ref[...] = ps
        a_dst_ref[...] = pd

    @pl.when(k > 0)
    def _():
        h2_ref[...] += ph
        a_src_ref[...] += ps
        a_dst_ref[...] += pd


def _project2(out1_flat, bias1, W2, watt_s, watt_d):
    nk = out1_flat.shape[0]
    return pl.pallas_call(
        _proj2_body,
        grid=(N_PAD // _BLK, nk),
        in_specs=[
            pl.BlockSpec((1, _BLK, HALF), lambda i, k: (k, i, 0)),
            pl.BlockSpec((1, 1, HALF), lambda i, k: (k, 0, 0)),
            pl.BlockSpec((1, HALF, HID), lambda i, k: (k, 0, 0)),
            pl.BlockSpec((1, HALF, 1), lambda i, k: (k, 0, 0)),
            pl.BlockSpec((1, HALF, 1), lambda i, k: (k, 0, 0)),
        ],
        out_specs=[
            pl.BlockSpec((_BLK, HID), lambda i, k: (i, 0)),
            pl.BlockSpec((_BLK, 1), lambda i, k: (i, 0)),
            pl.BlockSpec((_BLK, 1), lambda i, k: (i, 0)),
        ],
        out_shape=[
            jax.ShapeDtypeStruct((N_PAD, HID), jnp.float32),
            jax.ShapeDtypeStruct((N_PAD, 1), jnp.float32),
            jax.ShapeDtypeStruct((N_PAD, 1), jnp.float32),
        ],
    )(out1_flat, bias1.reshape(nk, 1, HALF), W2.reshape(nk, HALF, HID),
      watt_s.reshape(nk, HALF, 1), watt_d.reshape(nk, HALF, 1))


def _gat_sc_body(heads,
                 idx_hbm, asrcT_hbm, adstT_hbm, hflat_hbm,
                 out_hbm,
                 asrc_t, adst_t, den_s, rbf0, rbf1, rowsf0, rowsf1,
                 idxc0, idxc1, eac0, eac1,
                 sem_i0, sem_i1, sem_g0, sem_g1,
                 sem_s0, sem_s1, sem_d0, sem_d1,
                 sh_out, sh_den):
    csc = lax.axis_index("c")
    s = lax.axis_index("s")
    cbase = s * NCH
    row0 = s * ROWS_TEC

    zero16 = jnp.zeros((LANES,), jnp.float32)
    rbf = (rbf0, rbf1)
    rowsf = (rowsf0, rowsf1)
    idxc = (idxc0, idxc1)
    eac = (eac0, eac1)
    sem_i = (sem_i0, sem_i1)
    sem_g = (sem_g0, sem_g1)
    sem_s = (sem_s0, sem_s1)
    sem_d = (sem_d0, sem_d1)

    def issue_idx(c, b):
        pltpu.async_copy(idx_hbm.at[cbase + c], idxc[b], sem_i[b])

    def wait_idx(b):
        pltpu.make_async_copy(idx_hbm.at[cbase], idxc[b], sem_i[b]).wait()

    def issue_gather(b, u):
        pltpu.async_copy(hflat_hbm.at[u].at[idxc[b].at[0]], rbf[b], sem_g[b])

    def wait_gather(b, u):
        pltpu.make_async_copy(hflat_hbm.at[u].at[idxc[b].at[0]], rbf[b],
                              sem_g[b]).wait()

    def issue_scat(b):
        pltpu.async_copy(rowsf[b], sh_out.at[idxc[b].at[1]], sem_s[b],
                         add=True)

    def wait_scat(b):
        pltpu.make_async_copy(rowsf[b], sh_out.at[idxc[b].at[1]],
                              sem_s[b]).wait()

    def issue_den(b):
        pltpu.async_copy(eac[b], sh_den.at[idxc[b].at[1]], sem_d[b], add=True)

    def wait_den(b):
        pltpu.make_async_copy(eac[b], sh_den.at[idxc[b].at[1]],
                              sem_d[b]).wait()

    for hd in range(heads):
        u = hd * NC + csc

        def zrow(i, _):
            for j in range(HALF // LANES):
                rowsf0[i, pl.ds(j * LANES, LANES)] = zero16
            return 0
        lax.fori_loop(0, CHUNK, zrow, 0)
        for j in range(CHUNK // LANES):
            eac0[pl.ds(j * LANES, LANES)] = zero16
        for z in range(ROWS_TEC // CHUNK):
            pltpu.sync_copy(rowsf0, sh_out.at[pl.ds(row0 + z * CHUNK, CHUNK), :])
            pltpu.sync_copy(eac0, sh_den.at[pl.ds(row0 + z * CHUNK, CHUNK)])

        pltpu.sync_copy(asrcT_hbm.at[hd], asrc_t)
        pltpu.sync_copy(adstT_hbm.at[hd], adst_t)
        plsc.subcore_barrier()

        def logits(b):
            for j in range(CHUNK // LANES):
                jl = pl.ds(j * LANES, LANES)
                a = plsc.load_gather(asrc_t, [idxc[b][0, jl]])
                bl = plsc.load_gather(adst_t, [idxc[b][1, jl]])
                al = a + bl
                al = jnp.where(al >= 0.0, al, 0.2 * al)
                eac[b][jl] = jnp.exp(al)

        def scale(b):
            @plsc.parallel_loop(0, CHUNK, unroll=8)
            def _srow(r):
                av = plsc.load_gather(eac[b],
                                      [jnp.full((LANES,), r, jnp.int32)])
                for j in range(HALF // 32):
                    vi = rbf[b][r, pl.ds(j * LANES, LANES)]
                    v = plsc.bitcast(vi, jnp.bfloat16)
                    lo, hi = plsc.unpack(v, format=plsc.PackFormat.INTERLEAVED)
                    rowsf[b][r, pl.ds(j * 32, LANES)] = lo * av
                    rowsf[b][r, pl.ds(j * 32 + LANES, LANES)] = hi * av

        def half(c, b, pred_w, pred_n, pred_tail):
            b1 = 1 - b

            @pl.when(pred_w)
            def _():
                wait_den(b)
            logits(b)
            issue_den(b)

            @pl.when(pred_n)
            def _():
                wait_idx(b1)
                issue_gather(b1, u)

            wait_gather(b, u)

            @pl.when(pred_w)
            def _():
                wait_scat(b)
            scale(b)
            issue_scat(b)

            @pl.when(pred_tail)
            def _():
                issue_idx(c + 2, b)

        issue_idx(0, 0)
        wait_idx(0)
        issue_gather(0, u)
        issue_idx(1, 1)

        npair = NCH // 2

        def step(i, _):
            true_ = i >= 0
            last = npair - 1
            half(2 * i, 0, i >= 1, true_, i < last)
            half(2 * i + 1, 1, i >= 1, i < last, i < last)
            return 0
        lax.fori_loop(0, npair, step, 0)

        wait_scat(0)
        wait_scat(1)
        wait_den(0)
        wait_den(1)
        plsc.subcore_barrier()

        pltpu.sync_copy(sh_den.at[pl.ds(row0, ROWS_TEC)], den_s)

        def flush(z, _):
            r0 = row0 + z * CHUNK
            pltpu.sync_copy(sh_out.at[pl.ds(r0, CHUNK), :], rowsf0)

            @plsc.parallel_loop(0, CHUNK, unroll=4)
            def _norm(r):
                dv = plsc.load_gather(den_s, [jnp.full((LANES,),
                                                       z * CHUNK + r,
                                                       jnp.int32)])
                inv = 1.0 / (dv + 1e-16)
                for j in range(HALF // LANES):
                    jl = pl.ds(j * LANES, LANES)
                    rowsf0[r, jl] = rowsf0[r, jl] * inv
            pltpu.sync_copy(rowsf0, out_hbm.at[hd, csc, pl.ds(r0, CHUNK), :])
            return 0
        lax.fori_loop(0, ROWS_TEC // CHUNK, flush, 0)
        plsc.subcore_barrier()


def _gat_sc(idx_packed, asrcT, adstT, h_flat, heads):
    mesh = plsc.VectorSubcoreMesh(core_axis_name="c", subcore_axis_name="s",
                                  num_cores=NC, num_subcores=NS)
    return pl.kernel(
        functools.partial(_gat_sc_body, heads),
        out_type=jax.ShapeDtypeStruct((heads, NC, N_PAD, HALF), jnp.float32),
        mesh=mesh,
        compiler_params=pltpu.CompilerParams(needs_layout_passes=False, use_tc_tiling_on_sc=False),
        scratch_types=[
            pltpu.VMEM((N_PAD,), jnp.float32),
            pltpu.VMEM((N_PAD,), jnp.float32),
            pltpu.VMEM((ROWS_TEC,), jnp.float32),
            pltpu.VMEM((CHUNK, HALF // 2), jnp.int32),
            pltpu.VMEM((CHUNK, HALF // 2), jnp.int32),
            pltpu.VMEM((CHUNK, HALF), jnp.float32),
            pltpu.VMEM((CHUNK, HALF), jnp.float32),
            pltpu.VMEM((2, CHUNK), jnp.int32),
            pltpu.VMEM((2, CHUNK), jnp.int32),
            pltpu.VMEM((CHUNK,), jnp.float32),
            pltpu.VMEM((CHUNK,), jnp.float32),
            pltpu.SemaphoreType.DMA,
            pltpu.SemaphoreType.DMA,
            pltpu.SemaphoreType.DMA,
            pltpu.SemaphoreType.DMA,
            pltpu.SemaphoreType.DMA,
            pltpu.SemaphoreType.DMA,
            pltpu.SemaphoreType.DMA,
            pltpu.SemaphoreType.DMA,
            pltpu.VMEM_SHARED((N_PAD, HALF), jnp.float32),
            pltpu.VMEM_SHARED((N_PAD,), jnp.float32),
        ],
    )(idx_packed, asrcT, adstT, h_flat)


def _pack_cols(h_flat):
    sh = h_flat.shape
    v = h_flat.reshape(sh[:-1] + (sh[-1] // 32, 2, 16))
    v = v.swapaxes(-1, -2).reshape(sh[:-1] + (sh[-1] // 2, 2))
    return lax.bitcast_convert_type(v.astype(jnp.bfloat16), jnp.int32)


def _blockdiag(att, heads, d):
    eye = jnp.eye(heads, dtype=jnp.float32)
    return (att.reshape(heads, 1, d) * eye[:, :, None]).transpose(0, 2, 1).reshape(heads * d, heads)


def kernel(x, edge_index, W1, att_src1, att_dst1, bias1, W2, att_src2, att_dst2, bias2):
    idt = edge_index.dtype
    loop = jnp.arange(N_NODES, dtype=idt)
    n_pad_e = E_PAD - E_REAL
    pad_src = jnp.zeros((n_pad_e,), dtype=idt)
    pad_dst = (N_NODES + jnp.arange(n_pad_e, dtype=idt) % (N_PAD - N_NODES))
    src = jnp.concatenate([edge_index[0], loop, pad_src]).astype(jnp.int32)
    dst = jnp.concatenate([edge_index[1], loop, pad_dst]).astype(jnp.int32)
    idx_packed = jnp.stack([src.reshape(-1, CHUNK), dst.reshape(-1, CHUNK)],
                           axis=1)

    A_src1 = _blockdiag(att_src1, HEADS, HID)
    A_dst1 = _blockdiag(att_dst1, HEADS, HID)

    x_pad = jnp.pad(x, ((0, N_PAD - N_NODES), (0, 0)))

    h1, as1, ad1 = _project(x_pad, W1, A_src1, A_dst1, HEADS)
    h1_flat = _pack_cols(h1.reshape(N_PAD, HEADS, NC, HALF)
                           .transpose(1, 2, 0, 3).reshape(HEADS * NC, N_PAD, HALF))
    out1 = _gat_sc(idx_packed, as1.T, ad1.T, h1_flat, HEADS)

    watt_s = W2 @ att_src2.reshape(HID, 1)
    watt_d = W2 @ att_dst2.reshape(HID, 1)
    out1_flat = out1.reshape(HEADS * NC, N_PAD, HALF)
    h2, as2, ad2 = _project2(out1_flat, bias1, W2, watt_s, watt_d)
    h2_flat = _pack_cols(h2.reshape(N_PAD, NC, HALF).transpose(1, 0, 2))
    out2 = _gat_sc(idx_packed, as2.T, ad2.T, h2_flat, 1)

    out = jnp.concatenate([out2[0, 0, :N_NODES], out2[0, 1, :N_NODES]], axis=1)
    return out + bias2
